# baseline (speedup 1.0000x reference)
import functools

import jax
import jax.numpy as jnp
from jax import lax
from jax.experimental import pallas as pl
from jax.experimental.pallas import tpu as pltpu
from jax.experimental.pallas import tpu_sc as plsc

N = 10000
E = 320000
D = 128
U = 128
NC = 2
NS = 16
NW = NC * NS
EPW = E // NW
C = 80
CH = EPW // C
HD = D // NC
EPS = E // NS
C2 = 125
CH2 = EPS // C2
NB = 4
RPS = 624
ZR = 208

_mesh = plsc.VectorSubcoreMesh(core_axis_name="c", subcore_axis_name="s")


@functools.partial(
    pl.kernel,
    out_type=jax.ShapeDtypeStruct((NC * N,), jnp.float32),
    mesh=_mesh,
    scratch_types=[
        pltpu.VMEM((CH, C), jnp.int32),
        pltpu.VMEM((C,), jnp.float32),
        pltpu.VMEM((640,), jnp.float32),
        pltpu.VMEM_SHARED((N,), jnp.float32),
        pltpu.SemaphoreType.DMA,
        pltpu.SemaphoreType.DMA,
    ],
)
def _deg_phase(row_hbm, out_hbm, rowv, ones_v, zv, acc, d0, d1):
    c = lax.axis_index("c")
    s = lax.axis_index("s")
    w = s * NC + c
    pltpu.sync_copy(row_hbm.at[w], rowv)

    def fill_ones(i, carry):
        ones_v[pl.ds(pl.multiple_of(i * 16, 16), 16)] = jnp.full(
            (16,), 1.0, jnp.float32)
        return carry

    lax.fori_loop(0, C // 16, fill_ones, 0)

    def fill_z(i, carry):
        zv[pl.ds(pl.multiple_of(i * 16, 16), 16)] = jnp.zeros((16,), jnp.float32)
        return carry

    lax.fori_loop(0, 40, fill_z, 0)

    @pl.when(s < 15)
    def _():
        pltpu.sync_copy(zv.at[pl.ds(0, 624)],
                        acc.at[pl.ds(pl.multiple_of(s * 624, 8), 624)])

    @pl.when(s == 15)
    def _():
        pltpu.sync_copy(zv, acc.at[pl.ds(15 * 624, 640)])

    plsc.subcore_barrier()

    def body(p, carry):
        cp0 = pltpu.async_copy(ones_v, acc.at[rowv.at[2 * p]], d0, add=True)
        cp1 = pltpu.async_copy(ones_v, acc.at[rowv.at[2 * p + 1]], d1,
                               add=True)
        cp0.wait()
        cp1.wait()
        return carry

    lax.fori_loop(0, CH // 2, body, 0)
    @pl.when(CH % 2 == 1)
    def _():
        pltpu.sync_copy(ones_v, acc.at[rowv.at[CH - 1]], add=True)
    plsc.subcore_barrier()

    @pl.when(s < 15)
    def _():
        off = pl.multiple_of(s * 624, 8)
        off_o = pl.multiple_of(c * N + s * 624, 8)
        pltpu.sync_copy(acc.at[pl.ds(off, 624)], zv.at[pl.ds(0, 624)])
        pltpu.sync_copy(zv.at[pl.ds(0, 624)], out_hbm.at[pl.ds(off_o, 624)])

    @pl.when(s == 15)
    def _():
        off_o = pl.multiple_of(c * N + 15 * 624, 8)
        pltpu.sync_copy(acc.at[pl.ds(15 * 624, 640)], zv)
        pltpu.sync_copy(zv, out_hbm.at[pl.ds(off_o, 640)])


@functools.partial(
    pl.kernel,
    out_type=jax.ShapeDtypeStruct((NC, N, HD), jnp.float32),
    mesh=_mesh,
    scratch_types=[
        pltpu.VMEM((CH2, C2), jnp.int32),
        pltpu.VMEM((CH2, C2), jnp.int32),
        pltpu.VMEM((C2, HD), jnp.float32),
        pltpu.VMEM((C2, HD), jnp.float32),
        pltpu.VMEM((C2, HD), jnp.float32),
        pltpu.VMEM((C2, HD), jnp.float32),
        pltpu.VMEM((ZR, HD), jnp.float32),
        pltpu.VMEM_SHARED((N, HD), jnp.float32),
        pltpu.SemaphoreType.DMA,
        pltpu.SemaphoreType.DMA,
        pltpu.SemaphoreType.DMA,
        pltpu.SemaphoreType.DMA,
        pltpu.SemaphoreType.DMA,
        pltpu.SemaphoreType.DMA,
        pltpu.SemaphoreType.DMA,
        pltpu.SemaphoreType.DMA,
    ],
    compiler_params=pltpu.CompilerParams(use_tc_tiling_on_sc=False),
)
def _agg_phase(xs_hbm, col_hbm, row_hbm, out_hbm, colv, rowv, b0, b1, b2, b3,
               zb, acc, g0, g1, g2, g3, s0, s1, s2, s3):
    c = lax.axis_index("c")
    s = lax.axis_index("s")
    bufs = (b0, b1, b2, b3)
    gsems = (g0, g1, g2, g3)
    ssems = (s0, s1, s2, s3)
    table = xs_hbm.at[c]
    dummy = table.at[pl.ds(0, C2)]
    pltpu.sync_copy(col_hbm.at[s], colv)
    pltpu.sync_copy(row_hbm.at[s], rowv)

    for b in range(NB):
        pltpu.async_copy(table.at[colv.at[b]], bufs[b], gsems[b])

    def zrow(i, carry):
        for k in range(HD // 16):
            zb[i, pl.ds(16 * k, 16)] = jnp.zeros((16,), jnp.float32)
        return carry

    lax.fori_loop(0, ZR, zrow, 0)
    base = pl.multiple_of(s * RPS, 8)
    for t in range(RPS // ZR):
        pltpu.sync_copy(zb, acc.at[pl.ds(base + t * ZR, ZR)])
    @pl.when(s == NS - 1)
    def _():
        pltpu.sync_copy(zb.at[pl.ds(0, 16)], acc.at[pl.ds(N - 16, 16)])
    plsc.subcore_barrier()

    def body(q, carry):
        for b in range(NB):
            j = q * NB + b
            pltpu.make_async_copy(dummy, bufs[b], gsems[b]).wait()
            pltpu.async_copy(bufs[b], acc.at[rowv.at[j]], ssems[b], add=True)
        for b in range(NB):
            j = q * NB + b + NB
            @pl.when(j < CH2)
            def _():
                pltpu.make_async_copy(dummy, bufs[b], ssems[b]).wait()
                pltpu.async_copy(table.at[colv.at[j]], bufs[b], gsems[b])
        return carry

    lax.fori_loop(0, CH2 // NB, body, 0)
    for b in range(NB):
        pltpu.make_async_copy(dummy, bufs[b], ssems[b]).wait()
    plsc.subcore_barrier()

    for t in range(RPS // ZR):
        off = pl.multiple_of(s * RPS, 8) + t * ZR
        pltpu.sync_copy(acc.at[pl.ds(off, ZR)], zb)
        pltpu.sync_copy(zb, out_hbm.at[c, pl.ds(off, ZR)])
    @pl.when(s == NS - 1)
    def _():
        pltpu.sync_copy(acc.at[pl.ds(N - 16, 16)], zb.at[pl.ds(0, 16)])
        pltpu.sync_copy(zb.at[pl.ds(0, 16)], out_hbm.at[c, pl.ds(N - 16, 16)])


def _dis_from(degp_ref):
    dp = degp_ref[0] + degp_ref[1]
    return jnp.where(dp > 0, lax.rsqrt(jnp.maximum(dp, 1e-12)), 0.0)


def _xs_body(degp_ref, x_ref, o_ref):
    xsb = x_ref[...] * _dis_from(degp_ref)
    o_ref[0] = xsb[:, :HD]
    o_ref[1] = xsb[:, HD:]


RB = 10000

_xs_tc = pl.pallas_call(
    _xs_body,
    grid=(N // RB,),
    in_specs=[
        pl.BlockSpec((NC, RB, 1), lambda i: (0, i, 0)),
        pl.BlockSpec((RB, D), lambda i: (i, 0)),
    ],
    out_specs=pl.BlockSpec((NC, RB, HD), lambda i: (0, i, 0)),
    out_shape=jax.ShapeDtypeStruct((NC, N, HD), jnp.float32),
)


def _dense_body(p_ref, degp_ref, x_ref, w1_ref, b1_ref, w2_ref, b2_ref, o_ref):
    dis = _dis_from(degp_ref)
    agg = jnp.concatenate([p_ref[0], p_ref[1]], axis=1) * dis
    x = x_ref[...]
    y1 = jnp.dot(agg, w1_ref[...], preferred_element_type=jnp.float32) + b1_ref[...]
    y2 = jnp.dot(x * agg, w2_ref[...], preferred_element_type=jnp.float32) + b2_ref[...]
    o_ref[...] = (jnp.where(y1 >= 0, y1, 0.2 * y1)
                  + jnp.where(y2 >= 0, y2, 0.2 * y2))


RD = 10000

_dense_tc = pl.pallas_call(
    _dense_body,
    grid=(N // RD,),
    in_specs=[
        pl.BlockSpec((NC, RD, HD), lambda i: (0, i, 0)),
        pl.BlockSpec((NC, RD, 1), lambda i: (0, i, 0)),
        pl.BlockSpec((RD, D), lambda i: (i, 0)),
        pl.BlockSpec((D, U), lambda i: (0, 0)),
        pl.BlockSpec((1, U), lambda i: (0, 0)),
        pl.BlockSpec((D, U), lambda i: (0, 0)),
        pl.BlockSpec((1, U), lambda i: (0, 0)),
    ],
    out_specs=pl.BlockSpec((RD, U), lambda i: (i, 0)),
    out_shape=jax.ShapeDtypeStruct((N, U), jnp.float32),
)


def kernel(x, edge_index, W1, b1, W2, b2):
    row_w = edge_index[0].reshape(NW, CH, C)
    row_s = edge_index[0].reshape(NS, CH2, C2)
    col_s = edge_index[1].reshape(NS, CH2, C2)
    degp = _deg_phase(row_w)
    degp3 = degp.reshape(NC, N, 1)
    xs2 = _xs_tc(degp3, x)
    parts = _agg_phase(xs2, col_s, row_s)
    return _dense_tc(parts, degp3, x, W1, b1.reshape(1, U), W2, b2.reshape(1, U))


# speedup vs baseline: 1.0286x; 1.0286x over previous
import functools

import jax
import jax.numpy as jnp
from jax import lax
from jax.experimental import pallas as pl
from jax.experimental.pallas import tpu as pltpu
from jax.experimental.pallas import tpu_sc as plsc

N = 10000
E = 320000
D = 128
U = 128
NC = 2
NS = 16
NW = NC * NS
EPW = E // NW
C = 80
CH = EPW // C
HD = D // NC
EPS = E // NS
C2 = 125
CH2 = EPS // C2
NB = 4
RPS = 624
ZR = 208

_mesh = plsc.VectorSubcoreMesh(core_axis_name="c", subcore_axis_name="s")


@functools.partial(
    pl.kernel,
    out_type=jax.ShapeDtypeStruct((NC * N,), jnp.float32),
    mesh=_mesh,
    scratch_types=[
        pltpu.VMEM((CH, C), jnp.int32),
        pltpu.VMEM((C,), jnp.float32),
        pltpu.VMEM((640,), jnp.float32),
        pltpu.VMEM_SHARED((N,), jnp.float32),
        pltpu.SemaphoreType.DMA,
        pltpu.SemaphoreType.DMA,
    ],
)
def _deg_phase(row_hbm, out_hbm, rowv, ones_v, zv, acc, d0, d1):
    c = lax.axis_index("c")
    s = lax.axis_index("s")
    w = s * NC + c
    pltpu.sync_copy(row_hbm.at[w], rowv)

    def fill_ones(i, carry):
        ones_v[pl.ds(pl.multiple_of(i * 16, 16), 16)] = jnp.full(
            (16,), 1.0, jnp.float32)
        return carry

    lax.fori_loop(0, C // 16, fill_ones, 0)

    def fill_z(i, carry):
        zv[pl.ds(pl.multiple_of(i * 16, 16), 16)] = jnp.zeros((16,), jnp.float32)
        return carry

    lax.fori_loop(0, 40, fill_z, 0)

    @pl.when(s < 15)
    def _():
        pltpu.sync_copy(zv.at[pl.ds(0, 624)],
                        acc.at[pl.ds(pl.multiple_of(s * 624, 8), 624)])

    @pl.when(s == 15)
    def _():
        pltpu.sync_copy(zv, acc.at[pl.ds(15 * 624, 640)])

    plsc.subcore_barrier()

    def body(p, carry):
        cp0 = pltpu.async_copy(ones_v, acc.at[rowv.at[2 * p]], d0, add=True)
        cp1 = pltpu.async_copy(ones_v, acc.at[rowv.at[2 * p + 1]], d1,
                               add=True)
        cp0.wait()
        cp1.wait()
        return carry

    lax.fori_loop(0, CH // 2, body, 0)
    @pl.when(CH % 2 == 1)
    def _():
        pltpu.sync_copy(ones_v, acc.at[rowv.at[CH - 1]], add=True)
    plsc.subcore_barrier()

    @pl.when(s < 15)
    def _():
        off = pl.multiple_of(s * 624, 8)
        off_o = pl.multiple_of(c * N + s * 624, 8)
        pltpu.sync_copy(acc.at[pl.ds(off, 624)], zv.at[pl.ds(0, 624)])
        pltpu.sync_copy(zv.at[pl.ds(0, 624)], out_hbm.at[pl.ds(off_o, 624)])

    @pl.when(s == 15)
    def _():
        off_o = pl.multiple_of(c * N + 15 * 624, 8)
        pltpu.sync_copy(acc.at[pl.ds(15 * 624, 640)], zv)
        pltpu.sync_copy(zv, out_hbm.at[pl.ds(off_o, 640)])


@functools.partial(
    pl.kernel,
    out_type=jax.ShapeDtypeStruct((NC, N, HD), jnp.float32),
    mesh=_mesh,
    scratch_types=[
        pltpu.VMEM((CH2, C2), jnp.int32),
        pltpu.VMEM((CH2, C2), jnp.int32),
        pltpu.VMEM((C2, HD), jnp.float32),
        pltpu.VMEM((C2, HD), jnp.float32),
        pltpu.VMEM((C2, HD), jnp.float32),
        pltpu.VMEM((C2, HD), jnp.float32),
        pltpu.VMEM((ZR, HD), jnp.float32),
        pltpu.VMEM_SHARED((N, HD), jnp.float32),
        pltpu.SemaphoreType.DMA,
        pltpu.SemaphoreType.DMA,
        pltpu.SemaphoreType.DMA,
        pltpu.SemaphoreType.DMA,
        pltpu.SemaphoreType.DMA,
        pltpu.SemaphoreType.DMA,
        pltpu.SemaphoreType.DMA,
        pltpu.SemaphoreType.DMA,
    ],
    compiler_params=pltpu.CompilerParams(use_tc_tiling_on_sc=False),
)
def _agg_phase(xs_hbm, col_hbm, row_hbm, out_hbm, colv, rowv, b0, b1, b2, b3,
               zb, acc, g0, g1, g2, g3, s0, s1, s2, s3):
    c = lax.axis_index("c")
    s = lax.axis_index("s")
    bufs = (b0, b1, b2, b3)
    gsems = (g0, g1, g2, g3)
    ssems = (s0, s1, s2, s3)
    table = xs_hbm.at[c]
    dummy = table.at[pl.ds(0, C2)]
    pltpu.sync_copy(col_hbm.at[s], colv)
    pltpu.sync_copy(row_hbm.at[s], rowv)

    for b in range(NB):
        pltpu.async_copy(table.at[colv.at[b]], bufs[b], gsems[b])

    def zrow(i, carry):
        for k in range(HD // 16):
            zb[i, pl.ds(16 * k, 16)] = jnp.zeros((16,), jnp.float32)
        return carry

    lax.fori_loop(0, ZR, zrow, 0)
    base = pl.multiple_of(s * RPS, 8)
    for t in range(RPS // ZR):
        pltpu.sync_copy(zb, acc.at[pl.ds(base + t * ZR, ZR)])
    @pl.when(s == NS - 1)
    def _():
        pltpu.sync_copy(zb.at[pl.ds(0, 16)], acc.at[pl.ds(N - 16, 16)])
    plsc.subcore_barrier()

    def body(q, carry):
        for b in range(NB):
            j = q * NB + b
            pltpu.make_async_copy(dummy, bufs[b], gsems[b]).wait()
            pltpu.async_copy(bufs[b], acc.at[rowv.at[j]], ssems[b], add=True)
        for b in range(NB):
            j = q * NB + b + NB
            @pl.when(j < CH2)
            def _():
                pltpu.make_async_copy(dummy, bufs[b], ssems[b]).wait()
                pltpu.async_copy(table.at[colv.at[j]], bufs[b], gsems[b])
        return carry

    lax.fori_loop(0, CH2 // NB, body, 0)
    for b in range(NB):
        pltpu.make_async_copy(dummy, bufs[b], ssems[b]).wait()
    plsc.subcore_barrier()

    for t in range(RPS // ZR):
        off = pl.multiple_of(s * RPS, 8) + t * ZR
        pltpu.sync_copy(acc.at[pl.ds(off, ZR)], zb)
        pltpu.sync_copy(zb, out_hbm.at[c, pl.ds(off, ZR)])
    @pl.when(s == NS - 1)
    def _():
        pltpu.sync_copy(acc.at[pl.ds(N - 16, 16)], zb.at[pl.ds(0, 16)])
        pltpu.sync_copy(zb.at[pl.ds(0, 16)], out_hbm.at[c, pl.ds(N - 16, 16)])


def _dis_from(degp_ref):
    dp = degp_ref[0] + degp_ref[1]
    return jnp.where(dp > 0, lax.rsqrt(jnp.maximum(dp, 1e-12)), 0.0)


def _xs_body(degp_ref, x_ref, o_ref):
    xsb = x_ref[...] * _dis_from(degp_ref)
    o_ref[0] = xsb[:, :HD]
    o_ref[1] = xsb[:, HD:]


RB = 5000

_xs_tc = pl.pallas_call(
    _xs_body,
    grid=(N // RB,),
    in_specs=[
        pl.BlockSpec((NC, RB, 1), lambda i: (0, i, 0)),
        pl.BlockSpec((RB, D), lambda i: (i, 0)),
    ],
    out_specs=pl.BlockSpec((NC, RB, HD), lambda i: (0, i, 0)),
    out_shape=jax.ShapeDtypeStruct((NC, N, HD), jnp.float32),
)


def _dense_body(p_ref, degp_ref, x_ref, w1_ref, b1_ref, w2_ref, b2_ref, o_ref):
    dis = _dis_from(degp_ref)
    agg = jnp.concatenate([p_ref[0], p_ref[1]], axis=1) * dis
    x = x_ref[...]
    y1 = jnp.dot(agg, w1_ref[...], preferred_element_type=jnp.float32) + b1_ref[...]
    y2 = jnp.dot(x * agg, w2_ref[...], preferred_element_type=jnp.float32) + b2_ref[...]
    o_ref[...] = (jnp.where(y1 >= 0, y1, 0.2 * y1)
                  + jnp.where(y2 >= 0, y2, 0.2 * y2))


RD = 5000

_dense_tc = pl.pallas_call(
    _dense_body,
    grid=(N // RD,),
    in_specs=[
        pl.BlockSpec((NC, RD, HD), lambda i: (0, i, 0)),
        pl.BlockSpec((NC, RD, 1), lambda i: (0, i, 0)),
        pl.BlockSpec((RD, D), lambda i: (i, 0)),
        pl.BlockSpec((D, U), lambda i: (0, 0)),
        pl.BlockSpec((1, U), lambda i: (0, 0)),
        pl.BlockSpec((D, U), lambda i: (0, 0)),
        pl.BlockSpec((1, U), lambda i: (0, 0)),
    ],
    out_specs=pl.BlockSpec((RD, U), lambda i: (i, 0)),
    out_shape=jax.ShapeDtypeStruct((N, U), jnp.float32),
)


def kernel(x, edge_index, W1, b1, W2, b2):
    row_w = edge_index[0].reshape(NW, CH, C)
    row_s = edge_index[0].reshape(NS, CH2, C2)
    col_s = edge_index[1].reshape(NS, CH2, C2)
    degp = _deg_phase(row_w)
    degp3 = degp.reshape(NC, N, 1)
    xs2 = _xs_tc(degp3, x)
    parts = _agg_phase(xs2, col_s, row_s)
    return _dense_tc(parts, degp3, x, W1, b1.reshape(1, U), W2, b2.reshape(1, U))
